# Initial kernel scaffold; baseline (speedup 1.0000x reference)
import functools

import jax
import jax.numpy as jnp
from jax import lax
from jax.experimental import pallas as pl
from jax.experimental.pallas import tpu as pltpu
from jax.experimental.pallas import tpu_sc as plsc

NC = 2
NS = 16
NW = NC * NS
CH = 128


def _sc_deg_kernel(npad, nch):
    rps = npad // NS

    def body(dst_hbm, ones_hbm, zeros_hbm, out_hbm, dst_v, ones_v, acc, sem):
        c = lax.axis_index("c")
        s = lax.axis_index("s")
        w = c * NS + s
        pltpu.sync_copy(zeros_hbm.at[pl.ds(s * rps, rps)],
                        acc.at[pl.ds(s * rps, rps)])
        pltpu.sync_copy(ones_hbm, ones_v)
        pltpu.sync_copy(dst_hbm.at[w], dst_v)
        plsc.subcore_barrier()

        def step(j, carry):
            pltpu.sync_copy(ones_v, acc.at[dst_v.at[j]], add=True)
            return carry

        lax.fori_loop(0, nch, step, 0)
        plsc.subcore_barrier()
        pltpu.sync_copy(acc.at[pl.ds(s * rps, rps)],
                        out_hbm.at[c].at[pl.ds(s * rps, rps)])

    return pl.kernel(
        body,
        out_type=jax.ShapeDtypeStruct((NC, npad, 16), jnp.float32),
        mesh=plsc.VectorSubcoreMesh(core_axis_name="c", subcore_axis_name="s"),
        scratch_types=[
            pltpu.VMEM((nch, CH), jnp.int32),
            pltpu.VMEM((CH, 16), jnp.float32),
            pltpu.VMEM_SHARED((npad, 16), jnp.float32),
            pltpu.SemaphoreType.DMA,
        ],
    )


def _sc_edge_kernel(npad, nch, d):
    rps = npad // NS

    def body(y_hbm, src_hbm, dst_hbm, zeros_hbm, out_hbm,
             src_v, dst_v, rows_v, acc, sem):
        c = lax.axis_index("c")
        s = lax.axis_index("s")
        w = c * NS + s
        pltpu.sync_copy(zeros_hbm.at[pl.ds(s * rps, rps)],
                        acc.at[pl.ds(s * rps, rps)])
        pltpu.sync_copy(src_hbm.at[w], src_v)
        pltpu.sync_copy(dst_hbm.at[w], dst_v)
        plsc.subcore_barrier()

        def step(j, carry):
            pltpu.async_copy(y_hbm.at[src_v.at[j]], rows_v, sem).wait()
            pltpu.sync_copy(rows_v, acc.at[dst_v.at[j]], add=True)
            return carry

        lax.fori_loop(0, nch, step, 0)
        plsc.subcore_barrier()
        pltpu.sync_copy(acc.at[pl.ds(s * rps, rps)],
                        out_hbm.at[c].at[pl.ds(s * rps, rps)])

    return pl.kernel(
        body,
        out_type=jax.ShapeDtypeStruct((NC, npad, d), jnp.float32),
        mesh=plsc.VectorSubcoreMesh(core_axis_name="c", subcore_axis_name="s"),
        scratch_types=[
            pltpu.VMEM((nch, CH), jnp.int32),
            pltpu.VMEM((nch, CH), jnp.int32),
            pltpu.VMEM((CH, d), jnp.float32),
            pltpu.VMEM_SHARED((npad, d), jnp.float32),
            pltpu.SemaphoreType.DMA,
        ],
    )


def _tc_first(x_ref, w1_ref, dega_ref, y1_ref, dinv_ref):
    deg = dega_ref[0, :, 0:1] + dega_ref[1, :, 0:1] + 1.0
    dinv = lax.rsqrt(deg)
    h = jnp.dot(x_ref[...], w1_ref[...], preferred_element_type=jnp.float32)
    y1_ref[...] = h * dinv
    dinv_ref[...] = jnp.broadcast_to(dinv, y1_ref.shape)


def _tc_mid(p_ref, y1_ref, dinv_ref, b1_ref, w2_ref, y2_ref):
    ssum = p_ref[0] + p_ref[1] + y1_ref[...]
    h = jnp.maximum(dinv_ref[...] * ssum + b1_ref[...], 0.0)
    y2_ref[...] = jnp.dot(h, w2_ref[...],
                          preferred_element_type=jnp.float32) * dinv_ref[...]


def _tc_last(p_ref, y2_ref, dinv_ref, b2_ref, out_ref):
    out_ref[...] = dinv_ref[...] * (p_ref[0] + p_ref[1] + y2_ref[...]) + b2_ref[...]


def kernel(x, edge_index, W1, b1, W2, b2):
    n, d_in = x.shape
    d_hid = W1.shape[1]
    d_out = W2.shape[1]
    e = edge_index.shape[1]

    npad = ((n // CH) + 1) * CH
    et = ((e // NW + CH - 1) // CH) * CH
    nch = et // CH
    epad = et * NW

    src = edge_index[0].astype(jnp.int32)
    dst = edge_index[1].astype(jnp.int32)
    src_t = jnp.concatenate(
        [src, jnp.zeros((epad - e,), jnp.int32)]).reshape(NW, nch, CH)
    dst_t = jnp.concatenate(
        [dst, jnp.full((epad - e,), npad - 1, jnp.int32)]).reshape(NW, nch, CH)
    xp = jnp.zeros((npad, d_in), jnp.float32).at[:n].set(x)

    ones16 = jnp.ones((CH, 16), jnp.float32)
    zeros16 = jnp.zeros((npad, 16), jnp.float32)
    zeros_nd = jnp.zeros((npad, d_hid), jnp.float32)

    dega = _sc_deg_kernel(npad, nch)(dst_t, ones16, zeros16)

    grid = (npad // CH,)
    row_spec = pl.BlockSpec((CH, d_hid), lambda b: (b, 0))
    full_w = pl.BlockSpec((d_in, d_hid), lambda b: (0, 0))
    part_spec = pl.BlockSpec((NC, CH, d_hid), lambda b: (0, b, 0))
    bias_spec = pl.BlockSpec((1, d_hid), lambda b: (0, 0))

    y1, dinv = pl.pallas_call(
        _tc_first,
        grid=grid,
        in_specs=[pl.BlockSpec((CH, d_in), lambda b: (b, 0)), full_w,
                  pl.BlockSpec((NC, CH, 16), lambda b: (0, b, 0))],
        out_specs=[row_spec, row_spec],
        out_shape=[jax.ShapeDtypeStruct((npad, d_hid), jnp.float32),
                   jax.ShapeDtypeStruct((npad, d_hid), jnp.float32)],
    )(xp, W1, dega)

    edge_pass = _sc_edge_kernel(npad, nch, d_hid)
    p1 = edge_pass(y1, src_t, dst_t, zeros_nd)

    y2 = pl.pallas_call(
        _tc_mid,
        grid=grid,
        in_specs=[part_spec, row_spec, row_spec, bias_spec,
                  pl.BlockSpec((d_hid, d_out), lambda b: (0, 0))],
        out_specs=row_spec,
        out_shape=jax.ShapeDtypeStruct((npad, d_out), jnp.float32),
    )(p1, y1, dinv, b1.reshape(1, d_hid), W2)

    p2 = edge_pass(y2, src_t, dst_t, zeros_nd)

    out = pl.pallas_call(
        _tc_last,
        grid=grid,
        in_specs=[part_spec, row_spec, row_spec, bias_spec],
        out_specs=row_spec,
        out_shape=jax.ShapeDtypeStruct((npad, d_out), jnp.float32),
    )(p2, y2, dinv, b2.reshape(1, d_out))

    return out[:n]


# speedup vs baseline: 11.2499x; 11.2499x over previous
import functools

import jax
import jax.numpy as jnp
from jax import lax
from jax.experimental import pallas as pl
from jax.experimental.pallas import tpu as pltpu
from jax.experimental.pallas import tpu_sc as plsc

NC = 2
NS = 16
NW = NC * NS
CH = 128


def _sc_deg_kernel(npad, nch, d):
    rps = npad // NS

    def body(dst_hbm, ones_hbm, zeros_hbm, out_hbm, dst_v, ones_v, acc, sem):
        c = lax.axis_index("c")
        s = lax.axis_index("s")
        w = c * NS + s
        pltpu.sync_copy(zeros_hbm.at[pl.ds(s * rps, rps)],
                        acc.at[pl.ds(s * rps, rps)])
        pltpu.sync_copy(ones_hbm, ones_v)
        pltpu.sync_copy(dst_hbm.at[w], dst_v)
        plsc.subcore_barrier()

        def step(j, carry):
            pltpu.sync_copy(ones_v, acc.at[dst_v.at[j]], add=True)
            return carry

        lax.fori_loop(0, nch, step, 0)
        plsc.subcore_barrier()
        pltpu.sync_copy(acc.at[pl.ds(s * rps, rps)],
                        out_hbm.at[c].at[pl.ds(s * rps, rps)])

    return pl.kernel(
        body,
        out_type=jax.ShapeDtypeStruct((NC, npad, d), jnp.float32),
        mesh=plsc.VectorSubcoreMesh(core_axis_name="c", subcore_axis_name="s"),
        scratch_types=[
            pltpu.VMEM((nch, CH), jnp.int32),
            pltpu.VMEM((CH, d), jnp.float32),
            pltpu.VMEM_SHARED((npad, d), jnp.float32),
            pltpu.SemaphoreType.DMA,
        ],
    )


def _sc_edge_kernel(npad, nch, d):
    rps = npad // NS

    def body(y_hbm, src_hbm, dst_hbm, zeros_hbm, out_hbm,
             src_v, dst_v, rows_v, acc, sem):
        c = lax.axis_index("c")
        s = lax.axis_index("s")
        w = c * NS + s
        pltpu.sync_copy(zeros_hbm.at[pl.ds(s * rps, rps)],
                        acc.at[pl.ds(s * rps, rps)])
        pltpu.sync_copy(src_hbm.at[w], src_v)
        pltpu.sync_copy(dst_hbm.at[w], dst_v)
        plsc.subcore_barrier()

        def step(j, carry):
            pltpu.async_copy(y_hbm.at[src_v.at[j]], rows_v, sem).wait()
            pltpu.sync_copy(rows_v, acc.at[dst_v.at[j]], add=True)
            return carry

        lax.fori_loop(0, nch, step, 0)
        plsc.subcore_barrier()
        pltpu.sync_copy(acc.at[pl.ds(s * rps, rps)],
                        out_hbm.at[c].at[pl.ds(s * rps, rps)])

    return pl.kernel(
        body,
        out_type=jax.ShapeDtypeStruct((NC, npad, d), jnp.float32),
        mesh=plsc.VectorSubcoreMesh(core_axis_name="c", subcore_axis_name="s"),
        scratch_types=[
            pltpu.VMEM((nch, CH), jnp.int32),
            pltpu.VMEM((nch, CH), jnp.int32),
            pltpu.VMEM((CH, d), jnp.float32),
            pltpu.VMEM_SHARED((npad, d), jnp.float32),
            pltpu.SemaphoreType.DMA,
        ],
    )


def _tc_first(x_ref, w1_ref, dega_ref, y1_ref, dinv_ref):
    deg = dega_ref[0, :, 0:1] + dega_ref[1, :, 0:1] + 1.0
    dinv = lax.rsqrt(deg)
    h = jnp.dot(x_ref[...], w1_ref[...], preferred_element_type=jnp.float32)
    y1_ref[...] = h * dinv
    dinv_ref[...] = jnp.broadcast_to(dinv, y1_ref.shape)


def _tc_mid(p_ref, y1_ref, dinv_ref, b1_ref, w2_ref, y2_ref):
    ssum = p_ref[0] + p_ref[1] + y1_ref[...]
    h = jnp.maximum(dinv_ref[...] * ssum + b1_ref[...], 0.0)
    y2_ref[...] = jnp.dot(h, w2_ref[...],
                          preferred_element_type=jnp.float32) * dinv_ref[...]


def _tc_last(p_ref, y2_ref, dinv_ref, b2_ref, out_ref):
    out_ref[...] = dinv_ref[...] * (p_ref[0] + p_ref[1] + y2_ref[...]) + b2_ref[...]


def kernel(x, edge_index, W1, b1, W2, b2):
    n, d_in = x.shape
    d_hid = W1.shape[1]
    d_out = W2.shape[1]
    e = edge_index.shape[1]

    npad = ((n // CH) + 1) * CH
    et = ((e // NW + CH - 1) // CH) * CH
    nch = et // CH
    epad = et * NW

    src = edge_index[0].astype(jnp.int32)
    dst = edge_index[1].astype(jnp.int32)
    src_t = jnp.concatenate(
        [src, jnp.zeros((epad - e,), jnp.int32)]).reshape(NW, nch, CH)
    dst_t = jnp.concatenate(
        [dst, jnp.full((epad - e,), npad - 1, jnp.int32)]).reshape(NW, nch, CH)
    xp = jnp.zeros((npad, d_in), jnp.float32).at[:n].set(x)

    ones_nd = jnp.ones((CH, d_hid), jnp.float32)
    zeros_nd = jnp.zeros((npad, d_hid), jnp.float32)

    dega = _sc_deg_kernel(npad, nch, d_hid)(dst_t, ones_nd, zeros_nd)

    grid = (npad // CH,)
    row_spec = pl.BlockSpec((CH, d_hid), lambda b: (b, 0))
    full_w = pl.BlockSpec((d_in, d_hid), lambda b: (0, 0))
    part_spec = pl.BlockSpec((NC, CH, d_hid), lambda b: (0, b, 0))
    bias_spec = pl.BlockSpec((1, d_hid), lambda b: (0, 0))

    y1, dinv = pl.pallas_call(
        _tc_first,
        grid=grid,
        in_specs=[pl.BlockSpec((CH, d_in), lambda b: (b, 0)), full_w,
                  pl.BlockSpec((NC, CH, d_hid), lambda b: (0, b, 0))],
        out_specs=[row_spec, row_spec],
        out_shape=[jax.ShapeDtypeStruct((npad, d_hid), jnp.float32),
                   jax.ShapeDtypeStruct((npad, d_hid), jnp.float32)],
    )(xp, W1, dega)

    edge_pass = _sc_edge_kernel(npad, nch, d_hid)
    p1 = edge_pass(y1, src_t, dst_t, zeros_nd)

    y2 = pl.pallas_call(
        _tc_mid,
        grid=grid,
        in_specs=[part_spec, row_spec, row_spec, bias_spec,
                  pl.BlockSpec((d_hid, d_out), lambda b: (0, 0))],
        out_specs=row_spec,
        out_shape=jax.ShapeDtypeStruct((npad, d_out), jnp.float32),
    )(p1, y1, dinv, b1.reshape(1, d_hid), W2)

    p2 = edge_pass(y2, src_t, dst_t, zeros_nd)

    out = pl.pallas_call(
        _tc_last,
        grid=grid,
        in_specs=[part_spec, row_spec, row_spec, bias_spec],
        out_specs=row_spec,
        out_shape=jax.ShapeDtypeStruct((npad, d_out), jnp.float32),
    )(p2, y2, dinv, b2.reshape(1, d_out))

    return out[:n]
